# baseline (speedup 1.0000x reference)
import functools

import jax
import jax.numpy as jnp
from jax import lax
from jax.experimental import pallas as pl
from jax.experimental.pallas import tpu as pltpu
from jax.experimental.pallas import tpu_sc as plsc

N = 10000
E = 160000
R = 8
DIN = 384
H = 64
DOUT = 3
W2P = 16
NR = N * R
NC = 2
NS = 16
NW = NC * NS
NCH = 40
NPAIR = NCH // 2
CHUNK = 125
EPW = CHUNK * NCH
ZCH = 128
NP = 10240
RPS = NP // NS

_mesh = plsc.VectorSubcoreMesh(core_axis_name="c", subcore_axis_name="s")


@functools.partial(
    pl.kernel,
    out_type=jax.ShapeDtypeStruct((NW * NR,), jnp.float32),
    mesh=_mesh,
    scratch_types=[
        pltpu.VMEM((EPW,), jnp.int32),
        pltpu.VMEM((NR,), jnp.float32),
    ],
    compiler_params=pltpu.CompilerParams(needs_layout_passes=False),
)
def _sc_counts(cw_hbm, out_hbm, cw_v, c_v):
    cid = lax.axis_index("c")
    sid = lax.axis_index("s")
    wid = cid * NS + sid

    zero16 = jnp.zeros((16,), jnp.float32)

    def zbody(i, carry):
        c_v[pl.ds(i * 16, 16)] = zero16
        return carry

    lax.fori_loop(0, NR // 16, zbody, 0)

    pltpu.sync_copy(cw_hbm.at[wid], cw_v)
    ones16 = jnp.ones((16,), jnp.float32)

    def cbody(g, carry):
        idx = cw_v[pl.ds(g * 16, 16)]
        plsc.addupdate_scatter(c_v, [idx], ones16)
        return carry

    lax.fori_loop(0, EPW // 16, cbody, 0)
    rem = EPW - (EPW // 16) * 16
    if rem:
        idx = cw_v[pl.ds(EPW - 16, 16)]
        mask = lax.iota(jnp.int32, 16) >= (16 - rem)
        plsc.addupdate_scatter(c_v, [idx], ones16, mask=mask)

    pltpu.sync_copy(c_v, out_hbm.at[pl.ds(wid * NR, NR)])


def _make_sc_agg(width):
    nq = width // 16

    @functools.partial(
        pl.kernel,
        out_type=jax.ShapeDtypeStruct((NC, NP, width), jnp.float32),
        mesh=_mesh,
        scratch_types=[
            pltpu.VMEM((NCH, CHUNK), jnp.int32),
            pltpu.VMEM((NCH, CHUNK), jnp.int32),
            pltpu.VMEM((NCH, CHUNK), jnp.int32),
            pltpu.VMEM((ZCH, width), jnp.float32),
            pltpu.VMEM((ZCH, width), jnp.float32),
            pltpu.VMEM((CHUNK, 16), jnp.float32),
            pltpu.VMEM((CHUNK, 16), jnp.float32),
            pltpu.VMEM_SHARED((NP, width), jnp.float32),
            pltpu.SemaphoreType.DMA,
            pltpu.SemaphoreType.DMA,
            pltpu.SemaphoreType.DMA,
            pltpu.SemaphoreType.DMA,
        ],
        compiler_params=pltpu.CompilerParams(needs_layout_passes=False,
                                             use_tc_tiling_on_sc=False),
    )
    def body(z_hbm, wtab_hbm, gidx_hbm, cwidx_hbm, didx_hbm, out_hbm,
             gidx_v, cwidx_v, didx_v, rows_a, rows_b, wrows_a, wrows_b,
             acc_sh, sza, szb, swa, swb):
        cid = lax.axis_index("c")
        sid = lax.axis_index("s")
        wid = cid * NS + sid

        zero16 = jnp.zeros((16,), jnp.float32)

        def zrow(i, carry):
            for q in range(nq):
                rows_a[i, pl.ds(q * 16, 16)] = zero16
            return carry

        lax.fori_loop(0, ZCH, zrow, 0)
        base = sid * RPS
        for t in range(RPS // ZCH):
            pltpu.sync_copy(rows_a,
                            acc_sh.at[pl.ds(base + t * ZCH, ZCH)])

        pltpu.sync_copy(gidx_hbm.at[wid], gidx_v)
        pltpu.sync_copy(cwidx_hbm.at[wid], cwidx_v)
        pltpu.sync_copy(didx_hbm.at[wid], didx_v)
        plsc.subcore_barrier()

        def issue(ch, rv, wv, s1, s2):
            pltpu.async_copy(z_hbm.at[gidx_v.at[ch]],
                             rv.at[pl.ds(0, CHUNK)], s1)
            pltpu.async_copy(wtab_hbm.at[cwidx_v.at[ch]], wv, s2)

        def wait(rv, wv, s1, s2):
            pltpu.make_async_copy(z_hbm.at[gidx_v.at[0]],
                                  rv.at[pl.ds(0, CHUNK)], s1).wait()
            pltpu.make_async_copy(wtab_hbm.at[cwidx_v.at[0]], wv, s2).wait()

        def scale_scatter(ch, rv, wv):
            def scale(e, icarry):
                w = wv[e, pl.ds(0, 16)][0]
                for q in range(nq):
                    rv[e, pl.ds(q * 16, 16)] = rv[e, pl.ds(q * 16, 16)] * w
                return icarry

            lax.fori_loop(0, CHUNK, scale, 0, unroll=5)
            pltpu.sync_copy(rv.at[pl.ds(0, CHUNK)],
                            acc_sh.at[didx_v.at[ch]], add=True)

        issue(0, rows_a, wrows_a, sza, swa)

        def pair(p, carry):
            ch0 = 2 * p
            issue(ch0 + 1, rows_b, wrows_b, szb, swb)
            wait(rows_a, wrows_a, sza, swa)
            scale_scatter(ch0, rows_a, wrows_a)

            @pl.when(p < NPAIR - 1)
            def _():
                issue(ch0 + 2, rows_a, wrows_a, sza, swa)

            wait(rows_b, wrows_b, szb, swb)
            scale_scatter(ch0 + 1, rows_b, wrows_b)
            return carry

        lax.fori_loop(0, NPAIR, pair, 0)
        plsc.subcore_barrier()

        for t in range(RPS // ZCH):
            lo = base + t * ZCH
            pltpu.sync_copy(acc_sh.at[pl.ds(lo, ZCH)], rows_a)
            pltpu.sync_copy(rows_a, out_hbm.at[cid, pl.ds(lo, ZCH)])

    return body


_sc_agg64 = _make_sc_agg(H)

ZC2 = 200
RPT2 = NR // NS


@functools.partial(
    pl.kernel,
    out_type=jax.ShapeDtypeStruct((NC, NR, W2P), jnp.float32),
    mesh=_mesh,
    scratch_types=[
        pltpu.VMEM((NCH, CHUNK), jnp.int32),
        pltpu.VMEM((NCH, CHUNK), jnp.int32),
        pltpu.VMEM((ZC2, W2P), jnp.float32),
        pltpu.VMEM((ZC2, W2P), jnp.float32),
        pltpu.VMEM_SHARED((NR, W2P), jnp.float32),
        pltpu.SemaphoreType.DMA,
        pltpu.SemaphoreType.DMA,
    ],
    compiler_params=pltpu.CompilerParams(needs_layout_passes=False,
                                         use_tc_tiling_on_sc=False),
)
def _sc_agg16u(z_hbm, gidx_hbm, cwidx_hbm, out_hbm,
               gidx_v, cwidx_v, rows_a, rows_b, acc_sh, sza, szb):
    cid = lax.axis_index("c")
    sid = lax.axis_index("s")
    wid = cid * NS + sid

    zero16 = jnp.zeros((16,), jnp.float32)

    def zrow(i, carry):
        rows_a[i, pl.ds(0, 16)] = zero16
        return carry

    lax.fori_loop(0, ZC2, zrow, 0)
    base = sid * RPT2
    for t in range(RPT2 // ZC2):
        pltpu.sync_copy(rows_a, acc_sh.at[pl.ds(base + t * ZC2, ZC2)])

    pltpu.sync_copy(gidx_hbm.at[wid], gidx_v)
    pltpu.sync_copy(cwidx_hbm.at[wid], cwidx_v)
    plsc.subcore_barrier()

    def issue(ch, rv, s1):
        pltpu.async_copy(z_hbm.at[gidx_v.at[ch]], rv.at[pl.ds(0, CHUNK)], s1)

    def wait(rv, s1):
        pltpu.make_async_copy(z_hbm.at[gidx_v.at[0]],
                              rv.at[pl.ds(0, CHUNK)], s1).wait()

    def scatter(ch, rv):
        pltpu.sync_copy(rv.at[pl.ds(0, CHUNK)],
                        acc_sh.at[cwidx_v.at[ch]], add=True)

    issue(0, rows_a, sza)

    def pair(p, carry):
        ch0 = 2 * p
        issue(ch0 + 1, rows_b, szb)
        wait(rows_a, sza)
        scatter(ch0, rows_a)

        @pl.when(p < NPAIR - 1)
        def _():
            issue(ch0 + 2, rows_a, sza)

        wait(rows_b, szb)
        scatter(ch0 + 1, rows_b)
        return carry

    lax.fori_loop(0, NPAIR, pair, 0)
    plsc.subcore_barrier()

    for t in range(RPT2 // ZC2):
        lo = base + t * ZC2
        pltpu.sync_copy(acc_sh.at[pl.ds(lo, ZC2)], rows_a)
        pltpu.sync_copy(rows_a, out_hbm.at[cid, pl.ds(lo, ZC2)])


BN = 400
GRID = N // BN
CC = NR // GRID


def _tc1a_body(x_ref, w1_ref, r1_ref, b1_ref, z1_ref, base1_ref):
    xb = x_ref[...]
    z1_ref[...] = jnp.dot(xb, w1_ref[...], preferred_element_type=jnp.float32)
    base1_ref[...] = (
        jnp.dot(xb, r1_ref[...], preferred_element_type=jnp.float32) + b1_ref[...]
    )


_tc1a = pl.pallas_call(
    _tc1a_body,
    grid=(GRID,),
    in_specs=[
        pl.BlockSpec((BN, DIN), lambda i: (i, 0)),
        pl.BlockSpec((DIN, R * H), lambda i: (0, 0)),
        pl.BlockSpec((DIN, H), lambda i: (0, 0)),
        pl.BlockSpec((1, H), lambda i: (0, 0)),
    ],
    out_specs=(
        pl.BlockSpec((BN, R * H), lambda i: (i, 0)),
        pl.BlockSpec((BN, H), lambda i: (i, 0)),
    ),
    out_shape=(
        jax.ShapeDtypeStruct((N, R * H), jnp.float32),
        jax.ShapeDtypeStruct((N, H), jnp.float32),
    ),
)


def _tc1b_body(cnt_ref, wtab_ref):
    csum = jnp.sum(cnt_ref[...], axis=0)
    cinv = 1.0 / jnp.maximum(csum, 1.0)
    wtab_ref[...] = jnp.broadcast_to(cinv[:, None], (CC, 16))


_tc1b = pl.pallas_call(
    _tc1b_body,
    grid=(GRID,),
    in_specs=[pl.BlockSpec((NW, CC), lambda i: (0, i))],
    out_specs=pl.BlockSpec((CC, 16), lambda i: (i, 0)),
    out_shape=jax.ShapeDtypeStruct((NR, 16), jnp.float32),
)


def _tc2_body(b1_ref, agg_ref, w2_ref, r2_ref, b2_ref, z2_ref, base2_ref):
    h = jnp.maximum(b1_ref[...] + agg_ref[0] + agg_ref[1], 0.0)
    z2_ref[...] = jnp.dot(h, w2_ref[...], preferred_element_type=jnp.float32)
    base2_ref[...] = (
        jnp.dot(h, r2_ref[...], preferred_element_type=jnp.float32) + b2_ref[...]
    )


_tc2 = pl.pallas_call(
    _tc2_body,
    grid=(GRID,),
    in_specs=[
        pl.BlockSpec((BN, H), lambda i: (i, 0)),
        pl.BlockSpec((NC, BN, H), lambda i: (0, i, 0)),
        pl.BlockSpec((H, R * W2P), lambda i: (0, 0)),
        pl.BlockSpec((H, W2P), lambda i: (0, 0)),
        pl.BlockSpec((1, W2P), lambda i: (0, 0)),
    ],
    out_specs=(
        pl.BlockSpec((BN, R * W2P), lambda i: (i, 0)),
        pl.BlockSpec((BN, W2P), lambda i: (i, 0)),
    ),
    out_shape=(
        jax.ShapeDtypeStruct((N, R * W2P), jnp.float32),
        jax.ShapeDtypeStruct((N, W2P), jnp.float32),
    ),
)


def _tc3_body(b2_ref, agg_ref, wtab_ref, out_ref):
    s = (agg_ref[0] + agg_ref[1]) * wtab_ref[...]
    out_ref[...] = b2_ref[...] + jnp.sum(s.reshape(BN, R, W2P), axis=1)


_tc3 = pl.pallas_call(
    _tc3_body,
    grid=(GRID,),
    in_specs=[
        pl.BlockSpec((BN, W2P), lambda i: (i, 0)),
        pl.BlockSpec((NC, CC, W2P), lambda i: (0, i, 0)),
        pl.BlockSpec((CC, 16), lambda i: (i, 0)),
    ],
    out_specs=pl.BlockSpec((BN, W2P), lambda i: (i, 0)),
    out_shape=jax.ShapeDtypeStruct((N, W2P), jnp.float32),
)


def _impl(x, edge_index, edge_type, W1, root1, b1, W2, root2, b2):
    src = edge_index[0]
    dst = edge_index[1]
    et = edge_type

    gidx3 = (src * R + et).reshape(NW, NCH, CHUNK)
    cwidx = dst * R + et
    cwidx3 = cwidx.reshape(NW, NCH, CHUNK)
    didx3 = dst.reshape(NW, NCH, CHUNK)

    counts = _sc_counts(cwidx.reshape(NW, EPW)).reshape(NW, NR)

    W1cat = jnp.transpose(W1, (1, 0, 2)).reshape(DIN, R * H)
    z1, base1 = _tc1a(x, W1cat, root1, b1.reshape(1, H))
    wtab = _tc1b(counts)

    aggs1 = _sc_agg64(z1.reshape(NR, H), wtab, gidx3, cwidx3, didx3)

    W2p = jnp.pad(jnp.transpose(W2, (1, 0, 2)),
                  ((0, 0), (0, 0), (0, W2P - DOUT))).reshape(H, R * W2P)
    root2p = jnp.pad(root2, ((0, 0), (0, W2P - DOUT)))
    b2p = jnp.pad(b2, (0, W2P - DOUT)).reshape(1, W2P)
    z2, base2 = _tc2(base1, aggs1, W2p, root2p, b2p)

    aggs2 = _sc_agg16u(z2.reshape(NR, W2P), gidx3, cwidx3)

    out16 = _tc3(base2, aggs2, wtab)
    return out16[:, :DOUT]


kernel = jax.jit(_impl)


# speedup vs baseline: 1.2134x; 1.2134x over previous
import functools

import jax
import jax.numpy as jnp
from jax import lax
from jax.experimental import pallas as pl
from jax.experimental.pallas import tpu as pltpu
from jax.experimental.pallas import tpu_sc as plsc

N = 10000
E = 160000
R = 8
DIN = 384
H = 64
DOUT = 3
W2P = 16
NR = N * R
NC = 2
NS = 16
NW = NC * NS
NCH = 40
NPAIR = NCH // 2
CHUNK = 125
EPW = CHUNK * NCH
ZCH = 128
NP = 10240
RPS = NP // NS

_mesh = plsc.VectorSubcoreMesh(core_axis_name="c", subcore_axis_name="s")


@functools.partial(
    pl.kernel,
    out_type=jax.ShapeDtypeStruct((NW * NR,), jnp.float32),
    mesh=_mesh,
    scratch_types=[
        pltpu.VMEM((EPW,), jnp.int32),
        pltpu.VMEM((NR,), jnp.float32),
    ],
    compiler_params=pltpu.CompilerParams(needs_layout_passes=False),
)
def _sc_counts(cw_hbm, out_hbm, cw_v, c_v):
    cid = lax.axis_index("c")
    sid = lax.axis_index("s")
    wid = cid * NS + sid

    zero16 = jnp.zeros((16,), jnp.float32)

    def zbody(i, carry):
        c_v[pl.ds(i * 16, 16)] = zero16
        return carry

    lax.fori_loop(0, NR // 16, zbody, 0)

    pltpu.sync_copy(cw_hbm.at[wid], cw_v)
    ones16 = jnp.ones((16,), jnp.float32)

    def cbody(g, carry):
        idx = cw_v[pl.ds(g * 16, 16)]
        plsc.addupdate_scatter(c_v, [idx], ones16)
        return carry

    lax.fori_loop(0, EPW // 16, cbody, 0)
    rem = EPW - (EPW // 16) * 16
    if rem:
        idx = cw_v[pl.ds(EPW - 16, 16)]
        mask = lax.iota(jnp.int32, 16) >= (16 - rem)
        plsc.addupdate_scatter(c_v, [idx], ones16, mask=mask)

    pltpu.sync_copy(c_v, out_hbm.at[pl.ds(wid * NR, NR)])


def _make_sc_agg(width):
    nq = width // 16

    @functools.partial(
        pl.kernel,
        out_type=jax.ShapeDtypeStruct((NC, NP, width), jnp.float32),
        mesh=_mesh,
        scratch_types=[
            pltpu.VMEM((NCH, CHUNK), jnp.int32),
            pltpu.VMEM((NCH, CHUNK), jnp.int32),
            pltpu.VMEM((NCH, CHUNK), jnp.int32),
            pltpu.VMEM((ZCH, width), jnp.float32),
            pltpu.VMEM((ZCH, width), jnp.float32),
            pltpu.VMEM((CHUNK, 16), jnp.float32),
            pltpu.VMEM((CHUNK, 16), jnp.float32),
            pltpu.VMEM_SHARED((NP, width), jnp.float32),
            pltpu.SemaphoreType.DMA,
            pltpu.SemaphoreType.DMA,
            pltpu.SemaphoreType.DMA,
            pltpu.SemaphoreType.DMA,
        ],
        compiler_params=pltpu.CompilerParams(needs_layout_passes=False,
                                             use_tc_tiling_on_sc=False),
    )
    def body(z_hbm, wtab_hbm, gidx_hbm, cwidx_hbm, didx_hbm, out_hbm,
             gidx_v, cwidx_v, didx_v, rows_a, rows_b, wrows_a, wrows_b,
             acc_sh, sza, szb, swa, swb):
        cid = lax.axis_index("c")
        sid = lax.axis_index("s")
        wid = cid * NS + sid

        zero16 = jnp.zeros((16,), jnp.float32)

        def zrow(i, carry):
            for q in range(nq):
                rows_a[i, pl.ds(q * 16, 16)] = zero16
            return carry

        lax.fori_loop(0, ZCH, zrow, 0)
        base = sid * RPS
        for t in range(RPS // ZCH):
            pltpu.sync_copy(rows_a,
                            acc_sh.at[pl.ds(base + t * ZCH, ZCH)])

        pltpu.sync_copy(gidx_hbm.at[wid], gidx_v)
        pltpu.sync_copy(cwidx_hbm.at[wid], cwidx_v)
        pltpu.sync_copy(didx_hbm.at[wid], didx_v)
        plsc.subcore_barrier()

        def issue(ch, rv, wv, s1, s2):
            pltpu.async_copy(z_hbm.at[gidx_v.at[ch]],
                             rv.at[pl.ds(0, CHUNK)], s1)
            pltpu.async_copy(wtab_hbm.at[cwidx_v.at[ch]], wv, s2)

        def wait(rv, wv, s1, s2):
            pltpu.make_async_copy(z_hbm.at[gidx_v.at[0]],
                                  rv.at[pl.ds(0, CHUNK)], s1).wait()
            pltpu.make_async_copy(wtab_hbm.at[cwidx_v.at[0]], wv, s2).wait()

        def scale_scatter(ch, rv, wv):
            def scale(e, icarry):
                w = wv[e, pl.ds(0, 16)][0]
                for q in range(nq):
                    rv[e, pl.ds(q * 16, 16)] = rv[e, pl.ds(q * 16, 16)] * w
                return icarry

            lax.fori_loop(0, CHUNK, scale, 0, unroll=5)
            pltpu.sync_copy(rv.at[pl.ds(0, CHUNK)],
                            acc_sh.at[didx_v.at[ch]], add=True)

        issue(0, rows_a, wrows_a, sza, swa)

        def pair(p, carry):
            ch0 = 2 * p
            issue(ch0 + 1, rows_b, wrows_b, szb, swb)
            wait(rows_a, wrows_a, sza, swa)
            scale_scatter(ch0, rows_a, wrows_a)

            @pl.when(p < NPAIR - 1)
            def _():
                issue(ch0 + 2, rows_a, wrows_a, sza, swa)

            wait(rows_b, wrows_b, szb, swb)
            scale_scatter(ch0 + 1, rows_b, wrows_b)
            return carry

        lax.fori_loop(0, NPAIR, pair, 0)
        plsc.subcore_barrier()

        for t in range(RPS // ZCH):
            lo = base + t * ZCH
            pltpu.sync_copy(acc_sh.at[pl.ds(lo, ZCH)], rows_a)
            pltpu.sync_copy(rows_a, out_hbm.at[cid, pl.ds(lo, ZCH)])

    return body


_sc_agg64 = _make_sc_agg(H)
_sc_agg16 = _make_sc_agg(W2P)


BN = 400
GRID = N // BN
CC = NR // GRID


def _tc1a_body(x_ref, w1_ref, r1_ref, b1_ref, z1_ref, base1_ref):
    xb = x_ref[...]
    z1_ref[...] = jnp.dot(xb, w1_ref[...], preferred_element_type=jnp.float32)
    base1_ref[...] = (
        jnp.dot(xb, r1_ref[...], preferred_element_type=jnp.float32) + b1_ref[...]
    )


_tc1a = pl.pallas_call(
    _tc1a_body,
    grid=(GRID,),
    in_specs=[
        pl.BlockSpec((BN, DIN), lambda i: (i, 0)),
        pl.BlockSpec((DIN, R * H), lambda i: (0, 0)),
        pl.BlockSpec((DIN, H), lambda i: (0, 0)),
        pl.BlockSpec((1, H), lambda i: (0, 0)),
    ],
    out_specs=(
        pl.BlockSpec((BN, R * H), lambda i: (i, 0)),
        pl.BlockSpec((BN, H), lambda i: (i, 0)),
    ),
    out_shape=(
        jax.ShapeDtypeStruct((N, R * H), jnp.float32),
        jax.ShapeDtypeStruct((N, H), jnp.float32),
    ),
)


def _tc1b_body(cnt_ref, wtab_ref):
    csum = jnp.sum(cnt_ref[...], axis=0)
    cinv = 1.0 / jnp.maximum(csum, 1.0)
    wtab_ref[...] = jnp.broadcast_to(cinv[:, None], (CC, 16))


_tc1b = pl.pallas_call(
    _tc1b_body,
    grid=(GRID,),
    in_specs=[pl.BlockSpec((NW, CC), lambda i: (0, i))],
    out_specs=pl.BlockSpec((CC, 16), lambda i: (i, 0)),
    out_shape=jax.ShapeDtypeStruct((NR, 16), jnp.float32),
)


def _tc2_body(b1_ref, agg_ref, w2_ref, r2_ref, b2_ref, z2_ref, base2_ref):
    h = jnp.maximum(b1_ref[...] + agg_ref[0] + agg_ref[1], 0.0)
    z2_ref[...] = jnp.dot(h, w2_ref[...], preferred_element_type=jnp.float32)
    base2_ref[...] = (
        jnp.dot(h, r2_ref[...], preferred_element_type=jnp.float32) + b2_ref[...]
    )


_tc2 = pl.pallas_call(
    _tc2_body,
    grid=(GRID,),
    in_specs=[
        pl.BlockSpec((BN, H), lambda i: (i, 0)),
        pl.BlockSpec((NC, BN, H), lambda i: (0, i, 0)),
        pl.BlockSpec((H, R * W2P), lambda i: (0, 0)),
        pl.BlockSpec((H, W2P), lambda i: (0, 0)),
        pl.BlockSpec((1, W2P), lambda i: (0, 0)),
    ],
    out_specs=(
        pl.BlockSpec((BN, R * W2P), lambda i: (i, 0)),
        pl.BlockSpec((BN, W2P), lambda i: (i, 0)),
    ),
    out_shape=(
        jax.ShapeDtypeStruct((N, R * W2P), jnp.float32),
        jax.ShapeDtypeStruct((N, W2P), jnp.float32),
    ),
)


def _tc3_body(b2_ref, agg_ref, out_ref):
    out_ref[...] = b2_ref[...] + agg_ref[0] + agg_ref[1]


_tc3 = pl.pallas_call(
    _tc3_body,
    grid=(GRID,),
    in_specs=[
        pl.BlockSpec((BN, W2P), lambda i: (i, 0)),
        pl.BlockSpec((NC, BN, W2P), lambda i: (0, i, 0)),
    ],
    out_specs=pl.BlockSpec((BN, W2P), lambda i: (i, 0)),
    out_shape=jax.ShapeDtypeStruct((N, W2P), jnp.float32),
)


def _impl(x, edge_index, edge_type, W1, root1, b1, W2, root2, b2):
    src = edge_index[0]
    dst = edge_index[1]
    et = edge_type

    gidx3 = (src * R + et).reshape(NW, NCH, CHUNK)
    cwidx = dst * R + et
    cwidx3 = cwidx.reshape(NW, NCH, CHUNK)
    didx3 = dst.reshape(NW, NCH, CHUNK)

    counts = _sc_counts(cwidx.reshape(NW, EPW)).reshape(NW, NR)

    W1cat = jnp.transpose(W1, (1, 0, 2)).reshape(DIN, R * H)
    z1, base1 = _tc1a(x, W1cat, root1, b1.reshape(1, H))
    wtab = _tc1b(counts)

    aggs1 = _sc_agg64(z1.reshape(NR, H), wtab, gidx3, cwidx3, didx3)

    W2p = jnp.pad(jnp.transpose(W2, (1, 0, 2)),
                  ((0, 0), (0, 0), (0, W2P - DOUT))).reshape(H, R * W2P)
    root2p = jnp.pad(root2, ((0, 0), (0, W2P - DOUT)))
    b2p = jnp.pad(b2, (0, W2P - DOUT)).reshape(1, W2P)
    z2, base2 = _tc2(base1, aggs1, W2p, root2p, b2p)

    aggs2 = _sc_agg16(z2.reshape(NR, W2P), wtab, gidx3, cwidx3, didx3)

    out16 = _tc3(base2, aggs2)
    return out16[:, :DOUT]


kernel = jax.jit(_impl)


# speedup vs baseline: 1.4324x; 1.1805x over previous
import functools

import jax
import jax.numpy as jnp
from jax import lax
from jax.experimental import pallas as pl
from jax.experimental.pallas import tpu as pltpu
from jax.experimental.pallas import tpu_sc as plsc

N = 10000
E = 160000
R = 8
DIN = 384
H = 64
DOUT = 3
W2P = 16
NR = N * R
NC = 2
NS = 16
NW = NC * NS
NCH = 40
NPAIR = NCH // 2
CHUNK = 125
EPW = CHUNK * NCH
ZCH = 128
NP = 10240
RPS = NP // NS

_mesh = plsc.VectorSubcoreMesh(core_axis_name="c", subcore_axis_name="s")


CW = 128
CROWS = 640
RPT_C = CROWS // NS
KPT = RPT_C * CW
NRW = CROWS * CW


@functools.partial(
    pl.kernel,
    out_type=jax.ShapeDtypeStruct((NRW, 16), jnp.float32),
    mesh=_mesh,
    scratch_types=[
        pltpu.VMEM((EPW,), jnp.int32),
        pltpu.VMEM((CROWS, CW), jnp.float32),
        pltpu.VMEM((5, CW), jnp.int32),
        pltpu.VMEM((8 * CW, 16), jnp.float32),
        pltpu.VMEM_SHARED((CROWS, CW), jnp.float32),
    ],
    compiler_params=pltpu.CompilerParams(needs_layout_passes=False,
                                         use_tc_tiling_on_sc=False),
)
def _sc_counts(cw_hbm, wtab_hbm, cw_v, c_v, ridx_v, wst_v, c_sh):
    cid = lax.axis_index("c")
    sid = lax.axis_index("s")
    wid = cid * NS + sid

    zero16 = jnp.zeros((16,), jnp.float32)

    def zbody(i, carry):
        for g in range(CW // 16):
            c_v[i, pl.ds(g * 16, 16)] = zero16
        return carry

    lax.fori_loop(0, CROWS, zbody, 0)
    pltpu.sync_copy(c_v.at[pl.ds(0, RPT_C)],
                    c_sh.at[pl.ds(sid * RPT_C, RPT_C)])

    for t in range(5):
        for g in range(CW // 16):
            ridx_v[t, pl.ds(g * 16, 16)] = (
                t * CW + g * 16 + lax.iota(jnp.int32, 16)
            )

    pltpu.sync_copy(cw_hbm.at[wid], cw_v)
    ones16 = jnp.ones((16,), jnp.float32)

    def cbody(g, carry):
        k = cw_v[pl.ds(g * 16, 16)]
        plsc.addupdate_scatter(c_v, [k >> 7, k & 127], ones16)
        return carry

    lax.fori_loop(0, EPW // 16, cbody, 0)
    rem = EPW - (EPW // 16) * 16
    if rem:
        k = cw_v[pl.ds(EPW - 16, 16)]
        mask = lax.iota(jnp.int32, 16) >= (16 - rem)
        plsc.addupdate_scatter(c_v, [k >> 7, k & 127], ones16, mask=mask)

    plsc.subcore_barrier()
    for t in range(5):
        pltpu.sync_copy(c_v.at[pl.ds(t * CW, CW)],
                        c_sh.at[ridx_v.at[t]], add=True)
    plsc.subcore_barrier()

    rbase = sid * RPT_C
    pltpu.sync_copy(c_sh.at[pl.ds(rbase, RPT_C)], c_v.at[pl.ds(0, RPT_C)])

    for b in range(RPT_C // 8):
        def wrow8(ri, carry):
            for g in range(CW // 16):
                c16 = c_v[b * 8 + ri, pl.ds(g * 16, 16)]
                winv = 1.0 / jnp.maximum(c16, 1.0)
                for l in range(16):
                    wst_v[ri * CW + g * 16 + l, pl.ds(0, 16)] = jnp.full(
                        (16,), winv[l], jnp.float32)
            return carry

        lax.fori_loop(0, 8, wrow8, 0)
        pltpu.sync_copy(wst_v, wtab_hbm.at[pl.ds((rbase + b * 8) * CW, 8 * CW)])


def _make_sc_agg(width):
    nq = width // 16

    @functools.partial(
        pl.kernel,
        out_type=jax.ShapeDtypeStruct((NC, NP, width), jnp.float32),
        mesh=_mesh,
        scratch_types=[
            pltpu.VMEM((NCH, CHUNK), jnp.int32),
            pltpu.VMEM((NCH, CHUNK), jnp.int32),
            pltpu.VMEM((NCH, CHUNK), jnp.int32),
            pltpu.VMEM((ZCH, width), jnp.float32),
            pltpu.VMEM((ZCH, width), jnp.float32),
            pltpu.VMEM((CHUNK, 16), jnp.float32),
            pltpu.VMEM((CHUNK, 16), jnp.float32),
            pltpu.VMEM_SHARED((NP, width), jnp.float32),
            pltpu.SemaphoreType.DMA,
            pltpu.SemaphoreType.DMA,
            pltpu.SemaphoreType.DMA,
            pltpu.SemaphoreType.DMA,
        ],
        compiler_params=pltpu.CompilerParams(needs_layout_passes=False,
                                             use_tc_tiling_on_sc=False),
    )
    def body(z_hbm, wtab_hbm, gidx_hbm, cwidx_hbm, didx_hbm, out_hbm,
             gidx_v, cwidx_v, didx_v, rows_a, rows_b, wrows_a, wrows_b,
             acc_sh, sza, szb, swa, swb):
        cid = lax.axis_index("c")
        sid = lax.axis_index("s")
        wid = cid * NS + sid

        zero16 = jnp.zeros((16,), jnp.float32)

        def zrow(i, carry):
            for q in range(nq):
                rows_a[i, pl.ds(q * 16, 16)] = zero16
            return carry

        lax.fori_loop(0, ZCH, zrow, 0)
        base = sid * RPS
        for t in range(RPS // ZCH):
            pltpu.sync_copy(rows_a,
                            acc_sh.at[pl.ds(base + t * ZCH, ZCH)])

        pltpu.sync_copy(gidx_hbm.at[wid], gidx_v)
        pltpu.sync_copy(cwidx_hbm.at[wid], cwidx_v)
        pltpu.sync_copy(didx_hbm.at[wid], didx_v)
        plsc.subcore_barrier()

        def issue(ch, rv, wv, s1, s2):
            pltpu.async_copy(z_hbm.at[gidx_v.at[ch]],
                             rv.at[pl.ds(0, CHUNK)], s1)
            pltpu.async_copy(wtab_hbm.at[cwidx_v.at[ch]], wv, s2)

        def wait(rv, wv, s1, s2):
            pltpu.make_async_copy(z_hbm.at[gidx_v.at[0]],
                                  rv.at[pl.ds(0, CHUNK)], s1).wait()
            pltpu.make_async_copy(wtab_hbm.at[cwidx_v.at[0]], wv, s2).wait()

        def scale_scatter(ch, rv, wv):
            def scale(e, icarry):
                w = wv[e, pl.ds(0, 16)][0]
                for q in range(nq):
                    rv[e, pl.ds(q * 16, 16)] = rv[e, pl.ds(q * 16, 16)] * w
                return icarry

            lax.fori_loop(0, CHUNK, scale, 0, unroll=5)
            pltpu.sync_copy(rv.at[pl.ds(0, CHUNK)],
                            acc_sh.at[didx_v.at[ch]], add=True)

        issue(0, rows_a, wrows_a, sza, swa)

        def pair(p, carry):
            ch0 = 2 * p
            issue(ch0 + 1, rows_b, wrows_b, szb, swb)
            wait(rows_a, wrows_a, sza, swa)
            scale_scatter(ch0, rows_a, wrows_a)

            @pl.when(p < NPAIR - 1)
            def _():
                issue(ch0 + 2, rows_a, wrows_a, sza, swa)

            wait(rows_b, wrows_b, szb, swb)
            scale_scatter(ch0 + 1, rows_b, wrows_b)
            return carry

        lax.fori_loop(0, NPAIR, pair, 0)
        plsc.subcore_barrier()

        for t in range(RPS // ZCH):
            lo = base + t * ZCH
            pltpu.sync_copy(acc_sh.at[pl.ds(lo, ZCH)], rows_a)
            pltpu.sync_copy(rows_a, out_hbm.at[cid, pl.ds(lo, ZCH)])

    return body


_sc_agg64 = _make_sc_agg(H)
_sc_agg16 = _make_sc_agg(W2P)


BN = 400
GRID = N // BN
CC = NR // GRID


def _tc1a_body(x_ref, w1_ref, r1_ref, b1_ref, z1_ref, base1_ref):
    xb = x_ref[...]
    z1_ref[...] = jnp.dot(xb, w1_ref[...], preferred_element_type=jnp.float32)
    base1_ref[...] = (
        jnp.dot(xb, r1_ref[...], preferred_element_type=jnp.float32) + b1_ref[...]
    )


_tc1a = pl.pallas_call(
    _tc1a_body,
    grid=(GRID,),
    in_specs=[
        pl.BlockSpec((BN, DIN), lambda i: (i, 0)),
        pl.BlockSpec((DIN, R * H), lambda i: (0, 0)),
        pl.BlockSpec((DIN, H), lambda i: (0, 0)),
        pl.BlockSpec((1, H), lambda i: (0, 0)),
    ],
    out_specs=(
        pl.BlockSpec((BN, R * H), lambda i: (i, 0)),
        pl.BlockSpec((BN, H), lambda i: (i, 0)),
    ),
    out_shape=(
        jax.ShapeDtypeStruct((N, R * H), jnp.float32),
        jax.ShapeDtypeStruct((N, H), jnp.float32),
    ),
)


def _tc2_body(b1_ref, agg_ref, w2_ref, r2_ref, b2_ref, z2_ref, base2_ref):
    h = jnp.maximum(b1_ref[...] + agg_ref[0] + agg_ref[1], 0.0)
    z2_ref[...] = jnp.dot(h, w2_ref[...], preferred_element_type=jnp.float32)
    base2_ref[...] = (
        jnp.dot(h, r2_ref[...], preferred_element_type=jnp.float32) + b2_ref[...]
    )


_tc2 = pl.pallas_call(
    _tc2_body,
    grid=(GRID,),
    in_specs=[
        pl.BlockSpec((BN, H), lambda i: (i, 0)),
        pl.BlockSpec((NC, BN, H), lambda i: (0, i, 0)),
        pl.BlockSpec((H, R * W2P), lambda i: (0, 0)),
        pl.BlockSpec((H, W2P), lambda i: (0, 0)),
        pl.BlockSpec((1, W2P), lambda i: (0, 0)),
    ],
    out_specs=(
        pl.BlockSpec((BN, R * W2P), lambda i: (i, 0)),
        pl.BlockSpec((BN, W2P), lambda i: (i, 0)),
    ),
    out_shape=(
        jax.ShapeDtypeStruct((N, R * W2P), jnp.float32),
        jax.ShapeDtypeStruct((N, W2P), jnp.float32),
    ),
)


def _tc3_body(b2_ref, agg_ref, out_ref):
    out_ref[...] = b2_ref[...] + agg_ref[0] + agg_ref[1]


_tc3 = pl.pallas_call(
    _tc3_body,
    grid=(GRID,),
    in_specs=[
        pl.BlockSpec((BN, W2P), lambda i: (i, 0)),
        pl.BlockSpec((NC, BN, W2P), lambda i: (0, i, 0)),
    ],
    out_specs=pl.BlockSpec((BN, W2P), lambda i: (i, 0)),
    out_shape=jax.ShapeDtypeStruct((N, W2P), jnp.float32),
)


def _impl(x, edge_index, edge_type, W1, root1, b1, W2, root2, b2):
    src = edge_index[0]
    dst = edge_index[1]
    et = edge_type

    gidx3 = (src * R + et).reshape(NW, NCH, CHUNK)
    cwidx = dst * R + et
    cwidx3 = cwidx.reshape(NW, NCH, CHUNK)
    didx3 = dst.reshape(NW, NCH, CHUNK)

    wtab = _sc_counts(cwidx.reshape(NW, EPW))

    W1cat = jnp.transpose(W1, (1, 0, 2)).reshape(DIN, R * H)
    z1, base1 = _tc1a(x, W1cat, root1, b1.reshape(1, H))

    aggs1 = _sc_agg64(z1.reshape(NR, H), wtab, gidx3, cwidx3, didx3)

    W2p = jnp.pad(jnp.transpose(W2, (1, 0, 2)),
                  ((0, 0), (0, 0), (0, W2P - DOUT))).reshape(H, R * W2P)
    root2p = jnp.pad(root2, ((0, 0), (0, W2P - DOUT)))
    b2p = jnp.pad(b2, (0, W2P - DOUT)).reshape(1, W2P)
    z2, base2 = _tc2(base1, aggs1, W2p, root2p, b2p)

    aggs2 = _sc_agg16(z2.reshape(NR, W2P), wtab, gidx3, cwidx3, didx3)

    out16 = _tc3(base2, aggs2)
    return out16[:, :DOUT]


kernel = jax.jit(_impl)


# speedup vs baseline: 1.4441x; 1.0081x over previous
import functools

import jax
import jax.numpy as jnp
from jax import lax
from jax.experimental import pallas as pl
from jax.experimental.pallas import tpu as pltpu
from jax.experimental.pallas import tpu_sc as plsc

N = 10000
E = 160000
R = 8
DIN = 384
H = 64
DOUT = 3
W2P = 16
NR = N * R
NC = 2
NS = 16
NW = NC * NS
NCH = 40
NPAIR = NCH // 2
CHUNK = 125
EPW = CHUNK * NCH
ZCH = 128
NP = 10240
RPS = NP // NS

_mesh = plsc.VectorSubcoreMesh(core_axis_name="c", subcore_axis_name="s")


CW = 128
CROWS = 640
RPT_C = CROWS // NS
WPT = CROWS // NW
NRW = CROWS * CW


@functools.partial(
    pl.kernel,
    out_type=jax.ShapeDtypeStruct((NRW, 16), jnp.float32),
    mesh=_mesh,
    scratch_types=[
        pltpu.VMEM((EPW,), jnp.int32),
        pltpu.VMEM((CROWS, CW), jnp.float32),
        pltpu.VMEM((5, CW), jnp.int32),
        pltpu.VMEM((10 * CW, 16), jnp.float32),
        pltpu.VMEM_SHARED((CROWS, CW), jnp.float32),
    ],
    compiler_params=pltpu.CompilerParams(needs_layout_passes=False,
                                         use_tc_tiling_on_sc=False),
)
def _sc_counts(cw_hbm, wtab_hbm, cw_v, c_v, ridx_v, wst_v, c_sh):
    cid = lax.axis_index("c")
    sid = lax.axis_index("s")
    wid = cid * NS + sid

    zero16 = jnp.zeros((16,), jnp.float32)

    def zbody(i, carry):
        for g in range(CW // 16):
            c_v[i, pl.ds(g * 16, 16)] = zero16
        return carry

    lax.fori_loop(0, CROWS, zbody, 0)
    pltpu.sync_copy(c_v.at[pl.ds(0, RPT_C)],
                    c_sh.at[pl.ds(sid * RPT_C, RPT_C)])

    for t in range(5):
        for g in range(CW // 16):
            ridx_v[t, pl.ds(g * 16, 16)] = (
                t * CW + g * 16 + lax.iota(jnp.int32, 16)
            )

    ones16 = jnp.ones((16,), jnp.float32)
    for half in range(NC):
        pltpu.sync_copy(cw_hbm.at[sid + half * NS], cw_v)

        def cbody(g, carry):
            k = cw_v[pl.ds(g * 16, 16)]
            plsc.addupdate_scatter(c_v, [k >> 7, k & 127], ones16)
            return carry

        lax.fori_loop(0, EPW // 16, cbody, 0)
        rem = EPW - (EPW // 16) * 16
        if rem:
            k = cw_v[pl.ds(EPW - 16, 16)]
            mask = lax.iota(jnp.int32, 16) >= (16 - rem)
            plsc.addupdate_scatter(c_v, [k >> 7, k & 127], ones16, mask=mask)

    plsc.subcore_barrier()
    for t in range(5):
        pltpu.sync_copy(c_v.at[pl.ds(t * CW, CW)],
                        c_sh.at[ridx_v.at[t]], add=True)
    plsc.subcore_barrier()

    rbase = wid * WPT
    pltpu.sync_copy(c_sh.at[pl.ds(rbase, WPT)], c_v.at[pl.ds(0, WPT)])

    for b in range(WPT // 10):
        def wrow10(ri, carry):
            for g in range(CW // 16):
                c16 = c_v[b * 10 + ri, pl.ds(g * 16, 16)]
                winv = 1.0 / jnp.maximum(c16, 1.0)
                for l in range(16):
                    wst_v[ri * CW + g * 16 + l, pl.ds(0, 16)] = jnp.full(
                        (16,), winv[l], jnp.float32)
            return carry

        lax.fori_loop(0, 10, wrow10, 0)
        pltpu.sync_copy(wst_v,
                        wtab_hbm.at[pl.ds((rbase + b * 10) * CW, 10 * CW)])


def _make_sc_agg(width):
    nq = width // 16

    @functools.partial(
        pl.kernel,
        out_type=jax.ShapeDtypeStruct((NC, NP, width), jnp.float32),
        mesh=_mesh,
        scratch_types=[
            pltpu.VMEM((NCH, CHUNK), jnp.int32),
            pltpu.VMEM((NCH, CHUNK), jnp.int32),
            pltpu.VMEM((NCH, CHUNK), jnp.int32),
            pltpu.VMEM((ZCH, width), jnp.float32),
            pltpu.VMEM((ZCH, width), jnp.float32),
            pltpu.VMEM((CHUNK, 16), jnp.float32),
            pltpu.VMEM((CHUNK, 16), jnp.float32),
            pltpu.VMEM_SHARED((NP, width), jnp.float32),
            pltpu.SemaphoreType.DMA,
            pltpu.SemaphoreType.DMA,
            pltpu.SemaphoreType.DMA,
            pltpu.SemaphoreType.DMA,
        ],
        compiler_params=pltpu.CompilerParams(needs_layout_passes=False,
                                             use_tc_tiling_on_sc=False),
    )
    def body(z_hbm, wtab_hbm, gidx_hbm, cwidx_hbm, didx_hbm, out_hbm,
             gidx_v, cwidx_v, didx_v, rows_a, rows_b, wrows_a, wrows_b,
             acc_sh, sza, szb, swa, swb):
        cid = lax.axis_index("c")
        sid = lax.axis_index("s")
        wid = cid * NS + sid

        zero16 = jnp.zeros((16,), jnp.float32)

        def zrow(i, carry):
            for q in range(nq):
                rows_a[i, pl.ds(q * 16, 16)] = zero16
            return carry

        lax.fori_loop(0, ZCH, zrow, 0)
        base = sid * RPS
        for t in range(RPS // ZCH):
            pltpu.sync_copy(rows_a,
                            acc_sh.at[pl.ds(base + t * ZCH, ZCH)])

        pltpu.sync_copy(gidx_hbm.at[wid], gidx_v)
        pltpu.sync_copy(cwidx_hbm.at[wid], cwidx_v)
        pltpu.sync_copy(didx_hbm.at[wid], didx_v)
        plsc.subcore_barrier()

        def issue(ch, rv, wv, s1, s2):
            pltpu.async_copy(z_hbm.at[gidx_v.at[ch]],
                             rv.at[pl.ds(0, CHUNK)], s1)
            pltpu.async_copy(wtab_hbm.at[cwidx_v.at[ch]], wv, s2)

        def wait(rv, wv, s1, s2):
            pltpu.make_async_copy(z_hbm.at[gidx_v.at[0]],
                                  rv.at[pl.ds(0, CHUNK)], s1).wait()
            pltpu.make_async_copy(wtab_hbm.at[cwidx_v.at[0]], wv, s2).wait()

        def scale_scatter(ch, rv, wv):
            def scale(e, icarry):
                w = wv[e, pl.ds(0, 16)][0]
                for q in range(nq):
                    rv[e, pl.ds(q * 16, 16)] = rv[e, pl.ds(q * 16, 16)] * w
                return icarry

            lax.fori_loop(0, CHUNK, scale, 0, unroll=5)
            pltpu.sync_copy(rv.at[pl.ds(0, CHUNK)],
                            acc_sh.at[didx_v.at[ch]], add=True)

        issue(0, rows_a, wrows_a, sza, swa)

        def pair(p, carry):
            ch0 = 2 * p
            issue(ch0 + 1, rows_b, wrows_b, szb, swb)
            wait(rows_a, wrows_a, sza, swa)
            scale_scatter(ch0, rows_a, wrows_a)

            @pl.when(p < NPAIR - 1)
            def _():
                issue(ch0 + 2, rows_a, wrows_a, sza, swa)

            wait(rows_b, wrows_b, szb, swb)
            scale_scatter(ch0 + 1, rows_b, wrows_b)
            return carry

        lax.fori_loop(0, NPAIR, pair, 0)
        plsc.subcore_barrier()

        for t in range(RPS // ZCH):
            lo = base + t * ZCH
            pltpu.sync_copy(acc_sh.at[pl.ds(lo, ZCH)], rows_a)
            pltpu.sync_copy(rows_a, out_hbm.at[cid, pl.ds(lo, ZCH)])

    return body


_sc_agg64 = _make_sc_agg(H)
_sc_agg16 = _make_sc_agg(W2P)


BN = 400
GRID = N // BN
CC = NR // GRID


def _tc1a_body(x_ref, w1_ref, r1_ref, b1_ref, z1_ref, base1_ref):
    xb = x_ref[...]
    z1_ref[...] = jnp.dot(xb, w1_ref[...], preferred_element_type=jnp.float32)
    base1_ref[...] = (
        jnp.dot(xb, r1_ref[...], preferred_element_type=jnp.float32) + b1_ref[...]
    )


_tc1a = pl.pallas_call(
    _tc1a_body,
    grid=(GRID,),
    in_specs=[
        pl.BlockSpec((BN, DIN), lambda i: (i, 0)),
        pl.BlockSpec((DIN, R * H), lambda i: (0, 0)),
        pl.BlockSpec((DIN, H), lambda i: (0, 0)),
        pl.BlockSpec((1, H), lambda i: (0, 0)),
    ],
    out_specs=(
        pl.BlockSpec((BN, R * H), lambda i: (i, 0)),
        pl.BlockSpec((BN, H), lambda i: (i, 0)),
    ),
    out_shape=(
        jax.ShapeDtypeStruct((N, R * H), jnp.float32),
        jax.ShapeDtypeStruct((N, H), jnp.float32),
    ),
)


def _tc2_body(b1_ref, agg_ref, w2_ref, r2_ref, b2_ref, z2_ref, base2_ref):
    h = jnp.maximum(b1_ref[...] + agg_ref[0] + agg_ref[1], 0.0)
    z2_ref[...] = jnp.dot(h, w2_ref[...], preferred_element_type=jnp.float32)
    base2_ref[...] = (
        jnp.dot(h, r2_ref[...], preferred_element_type=jnp.float32) + b2_ref[...]
    )


_tc2 = pl.pallas_call(
    _tc2_body,
    grid=(GRID,),
    in_specs=[
        pl.BlockSpec((BN, H), lambda i: (i, 0)),
        pl.BlockSpec((NC, BN, H), lambda i: (0, i, 0)),
        pl.BlockSpec((H, R * W2P), lambda i: (0, 0)),
        pl.BlockSpec((H, W2P), lambda i: (0, 0)),
        pl.BlockSpec((1, W2P), lambda i: (0, 0)),
    ],
    out_specs=(
        pl.BlockSpec((BN, R * W2P), lambda i: (i, 0)),
        pl.BlockSpec((BN, W2P), lambda i: (i, 0)),
    ),
    out_shape=(
        jax.ShapeDtypeStruct((N, R * W2P), jnp.float32),
        jax.ShapeDtypeStruct((N, W2P), jnp.float32),
    ),
)


def _tc3_body(b2_ref, agg_ref, out_ref):
    out_ref[...] = b2_ref[...] + agg_ref[0] + agg_ref[1]


_tc3 = pl.pallas_call(
    _tc3_body,
    grid=(GRID,),
    in_specs=[
        pl.BlockSpec((BN, W2P), lambda i: (i, 0)),
        pl.BlockSpec((NC, BN, W2P), lambda i: (0, i, 0)),
    ],
    out_specs=pl.BlockSpec((BN, W2P), lambda i: (i, 0)),
    out_shape=jax.ShapeDtypeStruct((N, W2P), jnp.float32),
)


def _impl(x, edge_index, edge_type, W1, root1, b1, W2, root2, b2):
    src = edge_index[0]
    dst = edge_index[1]
    et = edge_type

    gidx3 = (src * R + et).reshape(NW, NCH, CHUNK)
    cwidx = dst * R + et
    cwidx3 = cwidx.reshape(NW, NCH, CHUNK)
    didx3 = dst.reshape(NW, NCH, CHUNK)

    wtab = _sc_counts(cwidx.reshape(NW, EPW))

    W1cat = jnp.transpose(W1, (1, 0, 2)).reshape(DIN, R * H)
    z1, base1 = _tc1a(x, W1cat, root1, b1.reshape(1, H))

    aggs1 = _sc_agg64(z1.reshape(NR, H), wtab, gidx3, cwidx3, didx3)

    W2p = jnp.pad(jnp.transpose(W2, (1, 0, 2)),
                  ((0, 0), (0, 0), (0, W2P - DOUT))).reshape(H, R * W2P)
    root2p = jnp.pad(root2, ((0, 0), (0, W2P - DOUT)))
    b2p = jnp.pad(b2, (0, W2P - DOUT)).reshape(1, W2P)
    z2, base2 = _tc2(base1, aggs1, W2p, root2p, b2p)

    aggs2 = _sc_agg16(z2.reshape(NR, W2P), wtab, gidx3, cwidx3, didx3)

    out16 = _tc3(base2, aggs2)
    return out16[:, :DOUT]


kernel = jax.jit(_impl)


# speedup vs baseline: 1.4595x; 1.0107x over previous
import functools

import jax
import jax.numpy as jnp
from jax import lax
from jax.experimental import pallas as pl
from jax.experimental.pallas import tpu as pltpu
from jax.experimental.pallas import tpu_sc as plsc

N = 10000
E = 160000
R = 8
DIN = 384
H = 64
DOUT = 3
W2P = 16
NR = N * R
NC = 2
NS = 16
NW = NC * NS
NCH = 40
NPAIR = NCH // 2
CHUNK = 125
EPW = CHUNK * NCH
ZCH = 128
NP = 10240
RPS = NP // NS

_mesh = plsc.VectorSubcoreMesh(core_axis_name="c", subcore_axis_name="s")


CW = 128
CROWS = 640
RPT_C = CROWS // NS
WPT = CROWS // NW
NRW = CROWS * CW


@functools.partial(
    pl.kernel,
    out_type=jax.ShapeDtypeStruct((NRW, 16), jnp.float32),
    mesh=_mesh,
    scratch_types=[
        pltpu.VMEM((EPW,), jnp.int32),
        pltpu.VMEM((CROWS, CW), jnp.float32),
        pltpu.VMEM((5, CW), jnp.int32),
        pltpu.VMEM((10 * CW, 16), jnp.float32),
        pltpu.VMEM_SHARED((CROWS, CW), jnp.float32),
    ],
    compiler_params=pltpu.CompilerParams(needs_layout_passes=False,
                                         use_tc_tiling_on_sc=False),
)
def _sc_counts(cw_hbm, wtab_hbm, cw_v, c_v, ridx_v, wst_v, c_sh):
    cid = lax.axis_index("c")
    sid = lax.axis_index("s")
    wid = cid * NS + sid

    zero16 = jnp.zeros((16,), jnp.float32)

    def zbody(i, carry):
        for g in range(CW // 16):
            c_v[i, pl.ds(g * 16, 16)] = zero16
        return carry

    lax.fori_loop(0, CROWS, zbody, 0)
    pltpu.sync_copy(c_v.at[pl.ds(0, RPT_C)],
                    c_sh.at[pl.ds(sid * RPT_C, RPT_C)])

    for t in range(5):
        for g in range(CW // 16):
            ridx_v[t, pl.ds(g * 16, 16)] = (
                t * CW + g * 16 + lax.iota(jnp.int32, 16)
            )

    ones16 = jnp.ones((16,), jnp.float32)
    for half in range(NC):
        pltpu.sync_copy(cw_hbm.at[sid + half * NS], cw_v)

        def cbody(g, carry):
            k = cw_v[pl.ds(g * 16, 16)]
            plsc.addupdate_scatter(c_v, [k >> 7, k & 127], ones16)
            return carry

        lax.fori_loop(0, EPW // 16, cbody, 0)
        rem = EPW - (EPW // 16) * 16
        if rem:
            k = cw_v[pl.ds(EPW - 16, 16)]
            mask = lax.iota(jnp.int32, 16) >= (16 - rem)
            plsc.addupdate_scatter(c_v, [k >> 7, k & 127], ones16, mask=mask)

    plsc.subcore_barrier()
    for t in range(5):
        pltpu.sync_copy(c_v.at[pl.ds(t * CW, CW)],
                        c_sh.at[ridx_v.at[t]], add=True)
    plsc.subcore_barrier()

    rbase = wid * WPT
    pltpu.sync_copy(c_sh.at[pl.ds(rbase, WPT)], c_v.at[pl.ds(0, WPT)])

    for b in range(WPT // 10):
        def wrow10(ri, carry):
            for g in range(CW // 16):
                c16 = c_v[b * 10 + ri, pl.ds(g * 16, 16)]
                winv = 1.0 / jnp.maximum(c16, 1.0)
                for l in range(16):
                    wst_v[ri * CW + g * 16 + l, pl.ds(0, 16)] = jnp.full(
                        (16,), winv[l], jnp.float32)
            return carry

        lax.fori_loop(0, 10, wrow10, 0)
        pltpu.sync_copy(wst_v,
                        wtab_hbm.at[pl.ds((rbase + b * 10) * CW, 10 * CW)])


def _make_sc_agg(width):
    nq = width // 16

    @functools.partial(
        pl.kernel,
        out_type=jax.ShapeDtypeStruct((NC, NP, width), jnp.float32),
        mesh=_mesh,
        scratch_types=[
            pltpu.VMEM((NCH, CHUNK), jnp.int32),
            pltpu.VMEM((NCH, CHUNK), jnp.int32),
            pltpu.VMEM((NCH, CHUNK), jnp.int32),
            pltpu.VMEM((ZCH, width), jnp.float32),
            pltpu.VMEM((ZCH, width), jnp.float32),
            pltpu.VMEM((CHUNK, 16), jnp.float32),
            pltpu.VMEM((CHUNK, 16), jnp.float32),
            pltpu.VMEM_SHARED((NP, width), jnp.float32),
            pltpu.SemaphoreType.DMA,
            pltpu.SemaphoreType.DMA,
            pltpu.SemaphoreType.DMA,
            pltpu.SemaphoreType.DMA,
        ],
        compiler_params=pltpu.CompilerParams(needs_layout_passes=False,
                                             use_tc_tiling_on_sc=False),
    )
    def body(z_hbm, wtab_hbm, gidx_hbm, cwidx_hbm, didx_hbm, out_hbm,
             gidx_v, cwidx_v, didx_v, rows_a, rows_b, wrows_a, wrows_b,
             acc_sh, sza, szb, swa, swb):
        cid = lax.axis_index("c")
        sid = lax.axis_index("s")
        wid = cid * NS + sid

        zero16 = jnp.zeros((16,), jnp.float32)

        def zrow(i, carry):
            for q in range(nq):
                rows_a[i, pl.ds(q * 16, 16)] = zero16
            return carry

        lax.fori_loop(0, ZCH, zrow, 0)
        base = sid * RPS
        for t in range(RPS // ZCH):
            pltpu.sync_copy(rows_a,
                            acc_sh.at[pl.ds(base + t * ZCH, ZCH)])

        pltpu.sync_copy(gidx_hbm.at[wid], gidx_v)
        pltpu.sync_copy(cwidx_hbm.at[wid], cwidx_v)
        pltpu.sync_copy(didx_hbm.at[wid], didx_v)
        plsc.subcore_barrier()

        def issue(ch, rv, wv, s1, s2):
            pltpu.async_copy(z_hbm.at[gidx_v.at[ch]],
                             rv.at[pl.ds(0, CHUNK)], s1)
            pltpu.async_copy(wtab_hbm.at[cwidx_v.at[ch]], wv, s2)

        def wait(rv, wv, s1, s2):
            pltpu.make_async_copy(z_hbm.at[gidx_v.at[0]],
                                  rv.at[pl.ds(0, CHUNK)], s1).wait()
            pltpu.make_async_copy(wtab_hbm.at[cwidx_v.at[0]], wv, s2).wait()

        def scale_scatter(ch, rv, wv):
            def scale(e, icarry):
                w = wv[e, pl.ds(0, 16)][0]
                for q in range(nq):
                    rv[e, pl.ds(q * 16, 16)] = rv[e, pl.ds(q * 16, 16)] * w
                return icarry

            lax.fori_loop(0, CHUNK, scale, 0, unroll=25)
            pltpu.sync_copy(rv.at[pl.ds(0, CHUNK)],
                            acc_sh.at[didx_v.at[ch]], add=True)

        issue(0, rows_a, wrows_a, sza, swa)

        def pair(p, carry):
            ch0 = 2 * p
            issue(ch0 + 1, rows_b, wrows_b, szb, swb)
            wait(rows_a, wrows_a, sza, swa)
            scale_scatter(ch0, rows_a, wrows_a)

            @pl.when(p < NPAIR - 1)
            def _():
                issue(ch0 + 2, rows_a, wrows_a, sza, swa)

            wait(rows_b, wrows_b, szb, swb)
            scale_scatter(ch0 + 1, rows_b, wrows_b)
            return carry

        lax.fori_loop(0, NPAIR, pair, 0)
        plsc.subcore_barrier()

        for t in range(RPS // ZCH):
            lo = base + t * ZCH
            pltpu.sync_copy(acc_sh.at[pl.ds(lo, ZCH)], rows_a)
            pltpu.sync_copy(rows_a, out_hbm.at[cid, pl.ds(lo, ZCH)])

    return body


_sc_agg64 = _make_sc_agg(H)
_sc_agg16 = _make_sc_agg(W2P)


BN = 400
GRID = N // BN
CC = NR // GRID


def _tc1a_body(x_ref, w1_ref, r1_ref, b1_ref, z1_ref, base1_ref):
    xb = x_ref[...]
    z1_ref[...] = jnp.dot(xb, w1_ref[...], preferred_element_type=jnp.float32)
    base1_ref[...] = (
        jnp.dot(xb, r1_ref[...], preferred_element_type=jnp.float32) + b1_ref[...]
    )


_tc1a = pl.pallas_call(
    _tc1a_body,
    grid=(GRID,),
    in_specs=[
        pl.BlockSpec((BN, DIN), lambda i: (i, 0)),
        pl.BlockSpec((DIN, R * H), lambda i: (0, 0)),
        pl.BlockSpec((DIN, H), lambda i: (0, 0)),
        pl.BlockSpec((1, H), lambda i: (0, 0)),
    ],
    out_specs=(
        pl.BlockSpec((BN, R * H), lambda i: (i, 0)),
        pl.BlockSpec((BN, H), lambda i: (i, 0)),
    ),
    out_shape=(
        jax.ShapeDtypeStruct((N, R * H), jnp.float32),
        jax.ShapeDtypeStruct((N, H), jnp.float32),
    ),
)


def _tc2_body(b1_ref, agg_ref, w2_ref, r2_ref, b2_ref, z2_ref, base2_ref):
    h = jnp.maximum(b1_ref[...] + agg_ref[0] + agg_ref[1], 0.0)
    z2_ref[...] = jnp.dot(h, w2_ref[...], preferred_element_type=jnp.float32)
    base2_ref[...] = (
        jnp.dot(h, r2_ref[...], preferred_element_type=jnp.float32) + b2_ref[...]
    )


_tc2 = pl.pallas_call(
    _tc2_body,
    grid=(GRID,),
    in_specs=[
        pl.BlockSpec((BN, H), lambda i: (i, 0)),
        pl.BlockSpec((NC, BN, H), lambda i: (0, i, 0)),
        pl.BlockSpec((H, R * W2P), lambda i: (0, 0)),
        pl.BlockSpec((H, W2P), lambda i: (0, 0)),
        pl.BlockSpec((1, W2P), lambda i: (0, 0)),
    ],
    out_specs=(
        pl.BlockSpec((BN, R * W2P), lambda i: (i, 0)),
        pl.BlockSpec((BN, W2P), lambda i: (i, 0)),
    ),
    out_shape=(
        jax.ShapeDtypeStruct((N, R * W2P), jnp.float32),
        jax.ShapeDtypeStruct((N, W2P), jnp.float32),
    ),
)


def _tc3_body(b2_ref, agg_ref, out_ref):
    out_ref[...] = b2_ref[...] + agg_ref[0] + agg_ref[1]


_tc3 = pl.pallas_call(
    _tc3_body,
    grid=(GRID,),
    in_specs=[
        pl.BlockSpec((BN, W2P), lambda i: (i, 0)),
        pl.BlockSpec((NC, BN, W2P), lambda i: (0, i, 0)),
    ],
    out_specs=pl.BlockSpec((BN, W2P), lambda i: (i, 0)),
    out_shape=jax.ShapeDtypeStruct((N, W2P), jnp.float32),
)


def _impl(x, edge_index, edge_type, W1, root1, b1, W2, root2, b2):
    src = edge_index[0]
    dst = edge_index[1]
    et = edge_type

    gidx3 = (src * R + et).reshape(NW, NCH, CHUNK)
    cwidx = dst * R + et
    cwidx3 = cwidx.reshape(NW, NCH, CHUNK)
    didx3 = dst.reshape(NW, NCH, CHUNK)

    wtab = _sc_counts(cwidx.reshape(NW, EPW))

    W1cat = jnp.transpose(W1, (1, 0, 2)).reshape(DIN, R * H)
    z1, base1 = _tc1a(x, W1cat, root1, b1.reshape(1, H))

    aggs1 = _sc_agg64(z1.reshape(NR, H), wtab, gidx3, cwidx3, didx3)

    W2p = jnp.pad(jnp.transpose(W2, (1, 0, 2)),
                  ((0, 0), (0, 0), (0, W2P - DOUT))).reshape(H, R * W2P)
    root2p = jnp.pad(root2, ((0, 0), (0, W2P - DOUT)))
    b2p = jnp.pad(b2, (0, W2P - DOUT)).reshape(1, W2P)
    z2, base2 = _tc2(base1, aggs1, W2p, root2p, b2p)

    aggs2 = _sc_agg16(z2.reshape(NR, W2P), wtab, gidx3, cwidx3, didx3)

    out16 = _tc3(base2, aggs2)
    return out16[:, :DOUT]


kernel = jax.jit(_impl)
